# baseline (speedup 1.0000x reference)
import functools
import math

import jax
import jax.numpy as jnp
from jax import lax
from jax.experimental import pallas as pl
from jax.experimental.pallas import tpu as pltpu
from jax.experimental.pallas import tpu_sc as plsc

PAD = 0
RATE = 0.1
NC = 2
NS = 16
LANES = 16


def _tc_total_body(h_ref, o_ref):
    i = pl.program_id(0)
    j = pl.program_id(1)

    @pl.when(jnp.logical_and(i == 0, j == 0))
    def _init():
        o_ref[...] = jnp.zeros_like(o_ref)

    o_ref[...] += jnp.sum(h_ref[...]).reshape(1, 1)


def _sc_gather_body(table_hbm, refs_hbm, out_hbm, refs_v, idx_v, vals_v,
                    acc_v, sem, *, rpw, v, c, tail0):
    wid = lax.axis_index("s") * NC + lax.axis_index("c")
    base = wid * rpw
    ng = rpw // LANES
    pltpu.sync_copy(refs_hbm.at[pl.ds(base, rpw)], refs_v)
    lane_iota = lax.iota(jnp.int32, LANES)
    for g in range(ng):
        r = refs_v[pl.ds(g * LANES, LANES)]
        i_vec = (base + g * LANES) + lane_iota
        idx_v[pl.ds(g * LANES, LANES)] = i_vec * v + r
        idx_v[pl.ds(rpw + g * LANES, LANES)] = i_vec * v
    pltpu.async_copy(table_hbm.at[idx_v], vals_v, sem).wait()
    acc = jnp.zeros((LANES,), jnp.float32)
    for g in range(ng):
        r = refs_v[pl.ds(g * LANES, LANES)]
        gold = vals_v[pl.ds(g * LANES, LANES)]
        h0 = vals_v[pl.ds(rpw + g * LANES, LANES)]
        tail = jnp.where(r != PAD, c * h0, tail0)
        acc = acc + (c - RATE) * gold + tail
    acc_v[...] = acc
    pltpu.sync_copy(acc_v, out_hbm.at[wid])


def kernel(hypotheses, references):
    B, S, V = hypotheses.shape
    N = B * S
    c = (1.0 - RATE) / (V - 2)
    k1 = (V - 2) * c * math.log(c) + RATE * math.log(RATE)
    nw = NC * NS
    rpw = N // nw

    VB = 6400
    NV = V // VB
    total = pl.pallas_call(
        _tc_total_body,
        grid=(B, NV),
        in_specs=[pl.BlockSpec((1, S, VB), lambda i, j: (i, 0, j))],
        out_specs=pl.BlockSpec((1, 1), lambda i, j: (0, 0)),
        out_shape=jax.ShapeDtypeStruct((1, 1), jnp.float32),
    )(hypotheses)

    table = hypotheses.reshape(N * V)
    refs_flat = references.astype(jnp.int32).reshape(N)
    mesh = plsc.VectorSubcoreMesh(core_axis_name="c", subcore_axis_name="s")
    sc_partials = pl.kernel(
        functools.partial(_sc_gather_body, rpw=rpw, v=V, c=c,
                          tail0=c * math.log(c)),
        out_type=jax.ShapeDtypeStruct((nw, LANES), jnp.float32),
        mesh=mesh,
        scratch_types=[
            pltpu.VMEM((rpw,), jnp.int32),
            pltpu.VMEM((2 * rpw,), jnp.int32),
            pltpu.VMEM((2 * rpw,), jnp.float32),
            pltpu.VMEM((LANES,), jnp.float32),
            pltpu.SemaphoreType.DMA,
        ],
    )(table, refs_flat)

    return N * k1 - c * total[0, 0] + jnp.sum(sc_partials)


# speedup vs baseline: 2.2402x; 2.2402x over previous
import functools
import math

import jax
import jax.numpy as jnp
from jax import lax
from jax.experimental import pallas as pl
from jax.experimental.pallas import tpu as pltpu
from jax.experimental.pallas import tpu_sc as plsc

PAD = 0
RATE = 0.1
NC = 2
NS = 16
LANES = 16
RPW = 64


def _tc_total_body(h_ref, o_ref):
    i = pl.program_id(0)
    j = pl.program_id(1)

    @pl.when(jnp.logical_and(i == 0, j == 0))
    def _init():
        o_ref[...] = jnp.zeros_like(o_ref)

    o_ref[...] += jnp.sum(h_ref[...]).reshape(1, 1)


def _sc_gather_body(hyp_hbm, refs_hbm, out_hbm, refs_v, tiles_v, h0tiles_v,
                    res_v, sem, hsem, *, s, c, tail0):
    wid = lax.axis_index("s") * NC + lax.axis_index("c")
    base = wid * RPW
    b = base // s
    s0 = base % s
    iota = lax.iota(jnp.int32, LANES)
    pltpu.sync_copy(refs_hbm.at[pl.ds(base, RPW)], refs_v)

    def rscalar(k):
        chunk = refs_v[pl.ds((k // LANES) * LANES, LANES)]
        return chunk[k % LANES]

    copies = []
    for k in range(RPW):
        r = rscalar(k)
        v128 = pl.multiple_of((r >> 7) << 7, 128)
        s8 = ((s0 + k) // 8) * 8
        copies.append(pltpu.async_copy(
            hyp_hbm.at[b, pl.ds(s8, 8), pl.ds(v128, 128)], tiles_v.at[k], sem))
    for g in range(RPW // 8):
        s8 = ((s0 + g * 8) // 8) * 8
        copies.append(pltpu.async_copy(
            hyp_hbm.at[b, pl.ds(s8, 8), pl.ds(0, 128)], h0tiles_v.at[g], hsem))
    for cp in copies:
        cp.wait()
    acc = jnp.zeros((LANES,), jnp.float32)
    tail0_vec = jnp.full((LANES,), tail0, jnp.float32)
    for k in range(RPW):
        r = rscalar(k)
        srow = (s0 + k) % 8
        off16 = pl.multiple_of(((r & 127) >> 4) << 4, 16)
        chunk = tiles_v[k, srow, pl.ds(off16, 16)]
        acc = acc + jnp.where(iota == (r & 15), (c - RATE) * chunk, 0.0)
        h0c = h0tiles_v[k // 8, srow, pl.ds(0, 16)]
        t0 = jnp.where(r != PAD, 1.0, 0.0)
        acc = acc + jnp.where(iota == 0,
                              t0 * (c * h0c) + (1.0 - t0) * tail0_vec, 0.0)
    res_v[...] = acc
    pltpu.sync_copy(res_v, out_hbm.at[wid])


def kernel(hypotheses, references):
    B, S, V = hypotheses.shape
    N = B * S
    c = (1.0 - RATE) / (V - 2)
    k1 = (V - 2) * c * math.log(c) + RATE * math.log(RATE)
    nw = NC * NS
    VB = 6400
    NV = V // VB
    refs_flat = references.astype(jnp.int32).reshape(N)

    total = pl.pallas_call(
        _tc_total_body,
        grid=(B, NV),
        in_specs=[pl.BlockSpec((1, S, VB), lambda i, j: (i, 0, j))],
        out_specs=pl.BlockSpec((1, 1), lambda i, j: (0, 0)),
        out_shape=jax.ShapeDtypeStruct((1, 1), jnp.float32),
    )(hypotheses)[0, 0]

    mesh = plsc.VectorSubcoreMesh(core_axis_name="c", subcore_axis_name="s")
    sc_partials = pl.kernel(
        functools.partial(_sc_gather_body, s=S, c=c, tail0=c * math.log(c)),
        out_type=jax.ShapeDtypeStruct((nw, LANES), jnp.float32),
        mesh=mesh,
        scratch_types=[
            pltpu.VMEM((RPW,), jnp.int32),
            pltpu.VMEM((RPW, 8, 128), jnp.float32),
            pltpu.VMEM((RPW // 8, 8, 128), jnp.float32),
            pltpu.VMEM((LANES,), jnp.float32),
            pltpu.SemaphoreType.DMA,
            pltpu.SemaphoreType.DMA,
        ],
        compiler_params=pltpu.CompilerParams(use_tc_tiling_on_sc=True),
    )(hypotheses, refs_flat)

    return N * k1 - c * total + jnp.sum(sc_partials)
